# baseline (speedup 1.0000x reference)
import functools

import jax
import jax.numpy as jnp
from jax import lax
from jax.experimental import pallas as pl
from jax.experimental.pallas import tpu as pltpu
from jax.experimental.pallas import tpu_sc as plsc

A = 50000
N = 32
B = 4
ROW = 16
CHUNK = 80
NCHUNKS = A // CHUNK
NW = 32
NQ = CHUNK // 16
NCPW = -(-NCHUNKS // NW)


def _sc_call(table, neigh_flat):
    mesh = plsc.VectorSubcoreMesh(core_axis_name="c", subcore_axis_name="s")

    @functools.partial(
        pl.kernel,
        out_type=jax.ShapeDtypeStruct((B * 3, N, A), jnp.float32),
        mesh=mesh,
        compiler_params=pltpu.CompilerParams(
            use_tc_tiling_on_sc=False, needs_layout_passes=False),
        scratch_types=[
            pltpu.VMEM((2, CHUNK * N), jnp.int32),
            pltpu.VMEM((2, CHUNK * N, ROW), jnp.float32),
            pltpu.VMEM((2, CHUNK, ROW), jnp.float32),
            pltpu.VMEM((B * 3, N, CHUNK), jnp.float32),
            pltpu.SemaphoreType.DMA,
            pltpu.SemaphoreType.DMA,
            pltpu.SemaphoreType.DMA,
            pltpu.SemaphoreType.DMA,
        ],
    )
    def k(table_hbm, neigh_hbm, out_hbm, idx_v, gath_v, cent_v, ob,
          gs0, gs1, cs0, cs1):
        cid = lax.axis_index("c")
        sid = lax.axis_index("s")
        wid = sid * 2 + cid
        nmine = jnp.where(wid < NCHUNKS - (NCHUNKS // NW) * NW, 1, 0) \
            + NCHUNKS // NW

        lane = lax.iota(jnp.int32, 16)
        lane_g = lane * N
        colv = [[lane * 0 + (3 * b + c) for c in range(3)] for b in range(B)]
        gsem = (gs0, gs1)
        csem = (cs0, cs1)

        def fetch(t, s):
            chunk = wid + NW * t
            a0 = chunk * CHUNK
            pltpu.sync_copy(neigh_hbm.at[pl.ds(a0 * N, CHUNK * N)],
                            idx_v.at[s])
            pltpu.async_copy(table_hbm.at[idx_v.at[s]], gath_v.at[s],
                             gsem[s])
            pltpu.async_copy(table_hbm.at[pl.ds(a0, CHUNK)],
                             cent_v.at[s], csem[s])

        def wait_fetch(s):
            pltpu.make_async_copy(table_hbm.at[idx_v.at[s]], gath_v.at[s],
                                  gsem[s]).wait()
            pltpu.make_async_copy(table_hbm.at[pl.ds(0, CHUNK)],
                                  cent_v.at[s], csem[s]).wait()

        def compute(t, s):
            chunk = wid + NW * t
            a0 = chunk * CHUNK
            gath = gath_v.at[s]
            cent = cent_v.at[s]

            def q_body(q, c2):
                crow = lane + q * 16
                cvec = [[plsc.load_gather(cent, [crow, colv[b][c]])
                         for c in range(3)] for b in range(B)]
                for n in range(N):
                    grow = lane_g + (q * (16 * N) + n)
                    for b in range(B):
                        for c in range(3):
                            g = plsc.load_gather(gath, [grow, colv[b][c]])
                            ob[b * 3 + c, n, pl.ds(q * 16, 16)] = \
                                g - cvec[b][c]
                return c2

            lax.fori_loop(0, NQ, q_body, 0)
            pltpu.sync_copy(ob, out_hbm.at[:, :, pl.ds(a0, CHUNK)])

        def step(t, s):
            @pl.when(t + 1 < nmine)
            def _():
                fetch(t + 1, 1 - s)
            wait_fetch(s)
            compute(t, s)

        fetch(0, 0)

        def pair_body(p, carry):
            t0 = 2 * p
            @pl.when(t0 < nmine)
            def _():
                step(t0, 0)

            @pl.when(t0 + 1 < nmine)
            def _():
                step(t0 + 1, 1)
            return carry

        lax.fori_loop(0, NCPW // 2, pair_body, 0)

    return k(table, neigh_flat)


def kernel(atoms, neighbors):
    table = jnp.transpose(atoms, (1, 0, 2)).reshape(A, 3 * B)
    table = jnp.pad(table, ((0, 0), (0, ROW - 3 * B)))
    neigh_flat = neighbors.astype(jnp.int32).reshape(-1)
    out = _sc_call(table, neigh_flat)
    return out.reshape(B, 3, N, A).transpose(0, 3, 2, 1)


# speedup vs baseline: 1.2257x; 1.2257x over previous
import functools

import jax
import jax.numpy as jnp
from jax import lax
from jax.experimental import pallas as pl
from jax.experimental.pallas import tpu as pltpu
from jax.experimental.pallas import tpu_sc as plsc

A = 50000
N = 32
B = 4
ROW = 13
CHUNK = 80
NCHUNKS = A // CHUNK
NW = 32
NQ = CHUNK // 16
NCPW = -(-NCHUNKS // NW)


def _sc_call(table, neigh_t):
    mesh = plsc.VectorSubcoreMesh(core_axis_name="c", subcore_axis_name="s")

    @functools.partial(
        pl.kernel,
        out_type=jax.ShapeDtypeStruct((B * 3, N, A), jnp.float32),
        mesh=mesh,
        compiler_params=pltpu.CompilerParams(
            use_tc_tiling_on_sc=False, needs_layout_passes=False),
        scratch_types=[
            pltpu.VMEM((2, N * CHUNK), jnp.int32),
            pltpu.VMEM((2, N * CHUNK, ROW), jnp.float32),
            pltpu.VMEM((2, CHUNK, ROW), jnp.float32),
            pltpu.VMEM((B * 3, N, CHUNK), jnp.float32),
            pltpu.SemaphoreType.DMA,
            pltpu.SemaphoreType.DMA,
            pltpu.SemaphoreType.DMA,
            pltpu.SemaphoreType.DMA,
        ],
    )
    def k(table_hbm, neigh_hbm, out_hbm, idx_v, gath_v, cent_v, ob,
          gs0, gs1, cs0, cs1):
        cid = lax.axis_index("c")
        sid = lax.axis_index("s")
        wid = sid * 2 + cid
        nmine = jnp.where(wid < NCHUNKS - (NCHUNKS // NW) * NW, 1, 0) \
            + NCHUNKS // NW

        lane = lax.iota(jnp.int32, 16)
        colv = [[lane * 0 + (3 * b + c) for c in range(3)] for b in range(B)]
        gsem = (gs0, gs1)
        csem = (cs0, cs1)

        def fetch(t, s):
            chunk = wid + NW * t
            a0 = chunk * CHUNK
            pltpu.sync_copy(
                neigh_hbm.at[pl.ds(chunk * (N * CHUNK), N * CHUNK)],
                idx_v.at[s])
            pltpu.async_copy(table_hbm.at[idx_v.at[s]], gath_v.at[s],
                             gsem[s])
            pltpu.async_copy(table_hbm.at[pl.ds(a0, CHUNK)],
                             cent_v.at[s], csem[s])

        def wait_fetch(s):
            pltpu.make_async_copy(table_hbm.at[idx_v.at[s]], gath_v.at[s],
                                  gsem[s]).wait()
            pltpu.make_async_copy(table_hbm.at[pl.ds(0, CHUNK)],
                                  cent_v.at[s], csem[s]).wait()

        def compute(t, s):
            chunk = wid + NW * t
            a0 = chunk * CHUNK
            gath = gath_v.at[s]
            cent = cent_v.at[s]

            def q_body(q, c2):
                crow = lane + q * 16
                cvec = [[plsc.load_gather(cent, [crow, colv[b][c]])
                         for c in range(3)] for b in range(B)]
                for n in range(N):
                    grow = crow + n * CHUNK
                    for b in range(B):
                        for c in range(3):
                            g = plsc.load_gather(gath, [grow, colv[b][c]])
                            ob[b * 3 + c, n, pl.ds(q * 16, 16)] = \
                                g - cvec[b][c]
                return c2

            lax.fori_loop(0, NQ, q_body, 0)
            pltpu.sync_copy(ob, out_hbm.at[:, :, pl.ds(a0, CHUNK)])

        def step(t, s):
            @pl.when(t + 1 < nmine)
            def _():
                fetch(t + 1, 1 - s)
            wait_fetch(s)
            compute(t, s)

        fetch(0, 0)

        def pair_body(p, carry):
            t0 = 2 * p
            @pl.when(t0 < nmine)
            def _():
                step(t0, 0)

            @pl.when(t0 + 1 < nmine)
            def _():
                step(t0 + 1, 1)
            return carry

        lax.fori_loop(0, NCPW // 2, pair_body, 0)

    return k(table, neigh_t)


def kernel(atoms, neighbors):
    table = jnp.transpose(atoms, (1, 0, 2)).reshape(A, 3 * B)
    table = jnp.pad(table, ((0, 0), (0, ROW - 3 * B)))
    neigh_t = (neighbors.astype(jnp.int32).T
               .reshape(N, NCHUNKS, CHUNK)
               .transpose(1, 0, 2)
               .reshape(-1))
    out = _sc_call(table, neigh_t)
    return out.reshape(B, 3, N, A).transpose(0, 3, 2, 1)


# speedup vs baseline: 1.3726x; 1.1199x over previous
import functools

import jax
import jax.numpy as jnp
from jax import lax
from jax.experimental import pallas as pl
from jax.experimental.pallas import tpu as pltpu
from jax.experimental.pallas import tpu_sc as plsc

A = 50000
N = 32
B = 4
PROW = 8
CHUNK = 80
NCHUNKS = A // CHUNK
NW = 32
NQ = CHUNK // 16
NCPW = -(-NCHUNKS // NW)


def _sc_call(table0, table1, neigh_t):
    mesh = plsc.VectorSubcoreMesh(core_axis_name="c", subcore_axis_name="s")

    @functools.partial(
        pl.kernel,
        out_type=jax.ShapeDtypeStruct((B * 3, N, A), jnp.float32),
        mesh=mesh,
        compiler_params=pltpu.CompilerParams(
            use_tc_tiling_on_sc=False, needs_layout_passes=False),
        scratch_types=[
            pltpu.VMEM((2, N * CHUNK), jnp.int32),
            pltpu.VMEM((2, 2, N * CHUNK, PROW), jnp.float32),
            pltpu.VMEM((2, 2, CHUNK, PROW), jnp.float32),
            pltpu.VMEM((B * 3, N, CHUNK), jnp.float32),
            pltpu.SemaphoreType.DMA,
            pltpu.SemaphoreType.DMA,
            pltpu.SemaphoreType.DMA,
            pltpu.SemaphoreType.DMA,
        ],
    )
    def k(t0_hbm, t1_hbm, neigh_hbm, out_hbm, idx_v, gath_v, cent_v, ob,
          gs0, gs1, cs0, cs1):
        cid = lax.axis_index("c")
        sid = lax.axis_index("s")
        wid = sid * 2 + cid
        nmine = jnp.where(wid < NCHUNKS - (NCHUNKS // NW) * NW, 1, 0) \
            + NCHUNKS // NW

        lane = lax.iota(jnp.int32, 16)
        tabs = (t0_hbm, t1_hbm)
        colv = [[lane * 0 + (3 * (b % 2) + c) for c in range(3)]
                for b in range(B)]
        gsem = (gs0, gs1)
        csem = (cs0, cs1)

        def fetch(t, s):
            chunk = wid + NW * t
            a0 = chunk * CHUNK
            pltpu.sync_copy(
                neigh_hbm.at[pl.ds(chunk * (N * CHUNK), N * CHUNK)],
                idx_v.at[s])
            for p in range(2):
                pltpu.async_copy(tabs[p].at[idx_v.at[s]], gath_v.at[s, p],
                                 gsem[s])
                pltpu.async_copy(tabs[p].at[pl.ds(a0, CHUNK)],
                                 cent_v.at[s, p], csem[s])

        def wait_fetch(s):
            for p in range(2):
                pltpu.make_async_copy(tabs[p].at[idx_v.at[s]],
                                      gath_v.at[s, p], gsem[s]).wait()
                pltpu.make_async_copy(tabs[p].at[pl.ds(0, CHUNK)],
                                      cent_v.at[s, p], csem[s]).wait()

        def compute(t, s):
            chunk = wid + NW * t
            a0 = chunk * CHUNK

            def q_body(q, c2):
                crow = lane + q * 16
                cvec = [[plsc.load_gather(cent_v.at[s, b // 2],
                                          [crow, colv[b][c]])
                         for c in range(3)] for b in range(B)]
                for n in range(N):
                    grow = crow + n * CHUNK
                    for b in range(B):
                        for c in range(3):
                            g = plsc.load_gather(gath_v.at[s, b // 2],
                                                 [grow, colv[b][c]])
                            ob[b * 3 + c, n, pl.ds(q * 16, 16)] = \
                                g - cvec[b][c]
                return c2

            lax.fori_loop(0, NQ, q_body, 0)
            pltpu.sync_copy(ob, out_hbm.at[:, :, pl.ds(a0, CHUNK)])

        def step(t, s):
            @pl.when(t + 1 < nmine)
            def _():
                fetch(t + 1, 1 - s)
            wait_fetch(s)
            compute(t, s)

        fetch(0, 0)

        def pair_body(p, carry):
            t0 = 2 * p
            @pl.when(t0 < nmine)
            def _():
                step(t0, 0)

            @pl.when(t0 + 1 < nmine)
            def _():
                step(t0 + 1, 1)
            return carry

        lax.fori_loop(0, NCPW // 2, pair_body, 0)

    return k(table0, table1, neigh_t)


def kernel(atoms, neighbors):
    t = jnp.transpose(atoms, (1, 0, 2)).reshape(A, 3 * B)
    table0 = jnp.pad(t[:, 0:6], ((0, 0), (0, PROW - 6)))
    table1 = jnp.pad(t[:, 6:12], ((0, 0), (0, PROW - 6)))
    neigh_t = (neighbors.astype(jnp.int32).T
               .reshape(N, NCHUNKS, CHUNK)
               .transpose(1, 0, 2)
               .reshape(-1))
    out = _sc_call(table0, table1, neigh_t)
    return out.reshape(B, 3, N, A).transpose(0, 3, 2, 1)
